# Initial kernel scaffold; baseline (speedup 1.0000x reference)
import jax
import jax.numpy as jnp
from jax.experimental import pallas as pl


def kernel(f, x, batch, edge_index, W_in, Wk, Wq, Wv, Rk, Rv, Ssh, Wo, Wr1, Wr2, Wr3):
    raise NotImplementedError("write your pallas kernel here")


# speedup vs baseline: 1.0493x; 1.0493x over previous
import functools

import jax
import jax.numpy as jnp
from jax.experimental import pallas as pl

N = 10000
E = 320000
D = 128
DK = 64
NB = 16
NL = 3
RMAX = 5.0
NSH = 9


def _mm_kernel(x_ref, w_ref, o_ref):
    o_ref[...] = jnp.dot(x_ref[...], w_ref[...], precision=jax.lax.Precision.DEFAULT, preferred_element_type=jnp.float32)


def _pallas_mm(x, w, bm=1000):
    m, k = x.shape
    n = w.shape[1]
    return pl.pallas_call(
        _mm_kernel,
        grid=(m // bm,),
        in_specs=[
            pl.BlockSpec((bm, k), lambda i: (i, 0)),
            pl.BlockSpec((k, n), lambda i: (0, 0)),
        ],
        out_specs=pl.BlockSpec((bm, n), lambda i: (i, 0)),
        out_shape=jax.ShapeDtypeStruct((m, n), jnp.float32),
    )(x, w)


def _sph_harm(d):
    x, y, z = d[:, 0], d[:, 1], d[:, 2]
    c0 = 0.28209479177387814
    c1 = 0.4886025119029199
    c2a = 1.0925484305920792
    c2b = 0.31539156525252005
    c2c = 0.5462742152960396
    sh0 = jnp.full_like(x, c0)[:, None]
    sh1 = jnp.stack([c1 * y, c1 * z, c1 * x], axis=1)
    sh2 = jnp.stack([c2a * x * y, c2a * y * z, c2b * (3.0 * z * z - 1.0), c2a * x * z, c2c * (x * x - y * y)], axis=1)
    return jnp.concatenate([sh0, sh1, sh2], axis=1)


def _rbf(dist):
    centers = jnp.linspace(0.0, RMAX, NB)
    width = RMAX / NB
    return jnp.exp(-((dist[:, None] - centers[None, :]) / width) ** 2)


def kernel(f, x, batch, edge_index, W_in, Wk, Wq, Wv, Rk, Rv, Ssh, Wo, Wr1, Wr2, Wr3):
    src = edge_index[0]
    dst = edge_index[1]
    h = _pallas_mm(f, W_in)
    vec = x[src] - x[dst]
    dist = jnp.sqrt(jnp.sum(vec * vec, axis=-1) + 1e-12)
    dhat = vec / dist[:, None]
    sh = _sph_harm(dhat)
    rb = _rbf(dist)
    for l in range(NL):
        kn = _pallas_mm(h, Wk[l])
        qn = _pallas_mm(h, Wq[l])
        vn = _pallas_mm(h, Wv[l])
        k = kn[src] * (rb @ Rk[l])
        v = vn[src] * (rb @ Rv[l]) + sh @ Ssh[l]
        logits = jnp.sum(k * qn[dst], axis=-1) / jnp.sqrt(float(DK))
        m = jax.ops.segment_max(logits, dst, num_segments=N)
        m = jnp.where(jnp.isfinite(m), m, 0.0)
        w = jnp.exp(logits - m[dst])
        denom = jax.ops.segment_sum(w, dst, num_segments=N) + 1e-9
        alpha = w / denom[dst]
        agg = jax.ops.segment_sum(alpha[:, None] * v, dst, num_segments=N)
        upd = jax.nn.silu(agg) @ Wo[l]
        h = h + upd
    out = ((h @ Wr1) * (h @ Wr2)) @ Wr3
    return out
